# baseline (speedup 1.0000x reference)
import functools

import jax
import jax.numpy as jnp
from jax import lax
from jax.experimental import pallas as pl
from jax.experimental.pallas import tpu as pltpu
from jax.experimental.pallas import tpu_sc as plsc

_NC = 2
_L = 16
_E = 64
_ZL = 0.001
_ROWS = 32768
_COLS = 2048
_BLK = 1024
_N = float(_ROWS * _COLS)


def _body(x_ref, p_ref, tk8_ref, probs_ref, loss_ref, topk_ref, acc_ref):
    step = pl.program_id(0)

    @pl.when(step == 0)
    def _init():
        acc_ref[...] = jnp.zeros_like(acc_ref)

    acc_ref[...] += jnp.sum(x_ref[...], axis=0, keepdims=True)

    @pl.when(step == pl.num_programs(0) - 1)
    def _finish():
        total = jnp.sum(acc_ref[...], keepdims=True)
        x = total / _N
        p = p_ref[...]
        sim = -((p - x) ** 2)
        m = jnp.max(sim, keepdims=True)
        e = jnp.exp(sim - m)
        denom = jnp.sum(e, keepdims=True)
        probs = e / denom

        idx = lax.broadcasted_iota(jnp.int32, (1, _E), 1)
        m1 = jnp.max(probs, keepdims=True)
        i1 = jnp.min(jnp.where(probs == m1, idx, _E), keepdims=True)
        rest = jnp.where(idx == i1, -jnp.inf, probs)
        m2 = jnp.max(rest, keepdims=True)
        i2 = jnp.min(jnp.where(rest == m2, idx, _E), keepdims=True)

        rows8 = lax.broadcasted_iota(jnp.int32, (8, 128), 0)
        tk8_ref[...] = jnp.where(rows8 == 0, i1, i2).astype(jnp.float32)
        probs_ref[...] = probs
        pm = jnp.sum(probs, keepdims=True) / _E
        loss_ref[...] = (pm - 1.0 / _E) ** 2 * _ZL
        k_iota = lax.broadcasted_iota(jnp.int32, (1, 2), 1)
        topk_ref[...] = jnp.where(k_iota == 0, i1, i2)


def _scatter_mask_body(tk8_hbm, mask_hbm, buft, outm, sem):
    wid = lax.axis_index("s") * _NC + lax.axis_index("c")

    @pl.when(wid == 0)
    def _run():
        pltpu.async_copy(tk8_hbm.at[pl.ds(0, 8), :], buft, sem).wait()
        i1 = buft[0, pl.ds(0, _L)]
        i2 = buft[1, pl.ds(0, _L)]
        one = jnp.ones((_L,), jnp.float32)
        zero = jnp.zeros((_L,), jnp.float32)
        for k in range(4):
            idx = (lax.iota(jnp.int32, _L) + _L * k).astype(jnp.float32)
            d1 = idx - i1
            d2 = idx - i2
            outm[pl.ds(_L * k, _L)] = (jnp.maximum(one - d1 * d1, zero)
                                       + jnp.maximum(one - d2 * d2, zero))
        pltpu.sync_copy(outm, mask_hbm)


_scatter_mask = functools.partial(
    pl.kernel,
    mesh=plsc.VectorSubcoreMesh(core_axis_name="c", subcore_axis_name="s"),
    out_type=jax.ShapeDtypeStruct((_E,), jnp.float32),
    scratch_types=[
        pltpu.VMEM((8, 128), jnp.float32),
        pltpu.VMEM((_E,), jnp.float32),
        pltpu.SemaphoreType.DMA,
    ],
)(_scatter_mask_body)


def kernel(wm_state, prototypes):
    wm = wm_state.reshape(_ROWS, _COLS)
    pt = prototypes.reshape(1, _E)
    grid = _ROWS // _BLK
    tk8, probs, loss, topk = pl.pallas_call(
        _body,
        grid=(grid,),
        in_specs=[
            pl.BlockSpec((_BLK, _COLS), lambda i: (i, 0)),
            pl.BlockSpec((1, _E), lambda i: (0, 0)),
        ],
        out_specs=[
            pl.BlockSpec((8, 128), lambda i: (0, 0)),
            pl.BlockSpec((1, _E), lambda i: (0, 0)),
            pl.BlockSpec((1, 1), lambda i: (0, 0)),
            pl.BlockSpec((1, 2), lambda i: (0, 0)),
        ],
        out_shape=[
            jax.ShapeDtypeStruct((8, 128), jnp.float32),
            jax.ShapeDtypeStruct((1, _E), jnp.float32),
            jax.ShapeDtypeStruct((1, 1), jnp.float32),
            jax.ShapeDtypeStruct((1, 2), jnp.int32),
        ],
        scratch_shapes=[pltpu.VMEM((1, _COLS), jnp.float32)],
    )(wm, pt)
    mask = _scatter_mask(tk8)
    return (mask, probs.reshape(_E),
            loss.reshape(()), topk.reshape(2))


# speedup vs baseline: 1.2215x; 1.2215x over previous
import jax
import jax.numpy as jnp
from jax import lax
from jax.experimental import pallas as pl
from jax.experimental.pallas import tpu as pltpu

_E = 64
_ZL = 0.001
_ROWS = 32768
_COLS = 2048
_BLK = 1024
_N = float(_ROWS * _COLS)


def _body(x_ref, p_ref, mask_ref, probs_ref, loss_ref, topk_ref, acc_ref):
    step = pl.program_id(0)

    @pl.when(step == 0)
    def _init():
        acc_ref[...] = jnp.zeros_like(acc_ref)

    acc_ref[...] += jnp.sum(x_ref[...], axis=0, keepdims=True)

    @pl.when(step == pl.num_programs(0) - 1)
    def _finish():
        total = jnp.sum(acc_ref[...], keepdims=True)
        x = total / _N
        p = p_ref[...]
        sim = -((p - x) ** 2)
        m = jnp.max(sim, keepdims=True)
        e = jnp.exp(sim - m)
        denom = jnp.sum(e, keepdims=True)
        probs = e / denom

        idx = lax.broadcasted_iota(jnp.int32, (1, _E), 1)
        m1 = jnp.max(probs, keepdims=True)
        i1 = jnp.min(jnp.where(probs == m1, idx, _E), keepdims=True)
        rest = jnp.where(idx == i1, -jnp.inf, probs)
        m2 = jnp.max(rest, keepdims=True)
        i2 = jnp.min(jnp.where(rest == m2, idx, _E), keepdims=True)

        mask_ref[...] = ((idx == i1) | (idx == i2)).astype(jnp.float32)
        probs_ref[...] = probs
        pm = jnp.sum(probs, keepdims=True) / _E
        loss_ref[...] = (pm - 1.0 / _E) ** 2 * _ZL
        k_iota = lax.broadcasted_iota(jnp.int32, (1, 2), 1)
        topk_ref[...] = jnp.where(k_iota == 0, i1, i2)


def kernel(wm_state, prototypes):
    wm = wm_state.reshape(_ROWS, _COLS)
    pt = prototypes.reshape(1, _E)
    grid = _ROWS // _BLK
    mask, probs, loss, topk = pl.pallas_call(
        _body,
        grid=(grid,),
        in_specs=[
            pl.BlockSpec((_BLK, _COLS), lambda i: (i, 0)),
            pl.BlockSpec((1, _E), lambda i: (0, 0)),
        ],
        out_specs=[
            pl.BlockSpec((1, _E), lambda i: (0, 0)),
            pl.BlockSpec((1, _E), lambda i: (0, 0)),
            pl.BlockSpec((1, 1), lambda i: (0, 0)),
            pl.BlockSpec((1, 2), lambda i: (0, 0)),
        ],
        out_shape=[
            jax.ShapeDtypeStruct((1, _E), jnp.float32),
            jax.ShapeDtypeStruct((1, _E), jnp.float32),
            jax.ShapeDtypeStruct((1, 1), jnp.float32),
            jax.ShapeDtypeStruct((1, 2), jnp.int32),
        ],
        scratch_shapes=[pltpu.VMEM((1, _COLS), jnp.float32)],
    )(wm, pt)
    return (mask.reshape(_E), probs.reshape(_E),
            loss.reshape(()), topk.reshape(2))
